# baseline (speedup 1.0000x reference)
import functools

import jax
import jax.numpy as jnp
from jax import lax
from jax.experimental import pallas as pl
from jax.experimental.pallas import tpu as pltpu
from jax.experimental.pallas import tpu_sc as plsc

N = 10000
E = 320000
D = 128
NPAD = 10240
NC = 2
NS = 16
NT = NC * NS
EPT = E // NT
C = 200
CHUNKS = EPT // C
STRIPE = NPAD // NS

_mesh = plsc.VectorSubcoreMesh(core_axis_name="c", subcore_axis_name="s")

_f32 = jnp.float32


@functools.partial(
    pl.kernel,
    mesh=_mesh,
    out_type=[
        jax.ShapeDtypeStruct((NC, NPAD, D), _f32),
        jax.ShapeDtypeStruct((NC, NPAD, D), _f32),
    ],
    scratch_types=[
        pltpu.VMEM((C,), jnp.int32),
        pltpu.VMEM((C, D), _f32),
        pltpu.VMEM_SHARED((NPAD, D), _f32),
    ],
)
def _sc_counts(nidx_hbm, eidx_hbm, ones_hbm, zeros_hbm,
               dcnt_hbm, bcnt_hbm, idx_v, ones_v, acc_sh):
    c = lax.axis_index("c")
    s = lax.axis_index("s")
    pltpu.sync_copy(ones_hbm, ones_v)
    base = (c * NS + s) * EPT

    for idx_hbm, out_hbm in ((nidx_hbm, dcnt_hbm), (eidx_hbm, bcnt_hbm)):
        pltpu.sync_copy(zeros_hbm.at[pl.ds(s * STRIPE, STRIPE)],
                        acc_sh.at[pl.ds(s * STRIPE, STRIPE)])
        plsc.subcore_barrier()

        @pl.loop(0, CHUNKS)
        def _(j):
            off = base + j * C
            pltpu.sync_copy(idx_hbm.at[pl.ds(off, C)], idx_v)
            pltpu.sync_copy(ones_v, acc_sh.at[idx_v], add=True)

        plsc.subcore_barrier()
        pltpu.sync_copy(acc_sh.at[pl.ds(s * STRIPE, STRIPE)],
                        out_hbm.at[c, pl.ds(s * STRIPE, STRIPE)])
        plsc.subcore_barrier()


CP = 160
GCP = 8
NGRP = 8
CHP = GCP * NGRP
CHPA = CHP + GCP
EPTP = CP * CHPA
EPAD = NT * EPTP
DUMP = NPAD - 1


@functools.partial(
    pl.kernel,
    mesh=_mesh,
    out_type=jax.ShapeDtypeStruct((NC, NPAD, D), _f32),
    scratch_types=[
        pltpu.VMEM((GCP * CP,), jnp.int32),
        pltpu.VMEM((GCP * CP,), jnp.int32),
        pltpu.VMEM((CP,), jnp.int32),
        pltpu.VMEM((CP,), jnp.int32),
        pltpu.VMEM((CP, D), _f32),
        pltpu.VMEM((CP, D), _f32),
        pltpu.VMEM_SHARED((NPAD, D), _f32),
        pltpu.SemaphoreType.DMA,
        pltpu.SemaphoreType.DMA,
        pltpu.SemaphoreType.DMA,
        pltpu.SemaphoreType.DMA,
    ],
)
def _sc_stage_pipe(feat_hbm, gidx_hbm, sidx_hbm, zeros_hbm, out_hbm,
                   gi_a, gi_b, si0, si1, r0, r1, acc_sh,
                   semg, sems, semi, semn):
    c = lax.axis_index("c")
    s = lax.axis_index("s")
    pltpu.sync_copy(zeros_hbm.at[pl.ds(s * STRIPE, STRIPE)],
                    acc_sh.at[pl.ds(s * STRIPE, STRIPE)])
    plsc.subcore_barrier()
    base = (c * NS + s) * EPTP
    rbufs = (r0, r1)
    sbufs = (si0, si1)

    def _gather(gi, k, rb):
        return pltpu.async_copy(feat_hbm.at[gi.at[pl.ds(k * CP, CP)]],
                                rb, semg)

    def _gwait(rb):
        pltpu.make_async_copy(feat_hbm.at[gi_a.at[pl.ds(0, CP)]],
                              rb, semg).wait()

    def _iwait(sb):
        pltpu.make_async_copy(sidx_hbm.at[pl.ds(base, CP)], sb, semi).wait()

    pltpu.sync_copy(gidx_hbm.at[pl.ds(base, GCP * CP)], gi_a)
    pltpu.async_copy(sidx_hbm.at[pl.ds(base, CP)], si0, semi)
    pltpu.async_copy(sidx_hbm.at[pl.ds(base + CP, CP)], si1, semi)
    _gather(gi_a, 0, r0)
    _gather(gi_a, 1, r1)

    def _group(g, gi_cur, gi_nxt):
        gbase = base + g * (GCP * CP)
        nh = pltpu.async_copy(gidx_hbm.at[pl.ds(gbase + GCP * CP, GCP * CP)],
                              gi_nxt, semn)
        for k in range(GCP):
            rb = rbufs[k % 2]
            sb = sbufs[k % 2]
            _gwait(rb)
            _iwait(sb)
            sh = pltpu.async_copy(rb, acc_sh.at[sb], sems, add=True)
            sh.wait()
            if k == GCP - 2:
                nh.wait()
            k2 = k + 2
            if k2 < GCP:
                _gather(gi_cur, k2, rb)
            else:
                _gather(gi_nxt, k2 - GCP, rb)
            pltpu.async_copy(sidx_hbm.at[pl.ds(gbase + k2 * CP, CP)],
                             sb, semi)

    @pl.loop(0, NGRP // 2)
    def _(p):
        _group(2 * p, gi_a, gi_b)
        _group(2 * p + 1, gi_b, gi_a)

    _gwait(r0)
    _gwait(r1)
    _iwait(si0)
    _iwait(si1)

    plsc.subcore_barrier()
    pltpu.sync_copy(acc_sh.at[pl.ds(s * STRIPE, STRIPE)],
                    out_hbm.at[c, pl.ds(s * STRIPE, STRIPE)])


_RB = 400
_GRID = N // _RB


def _inv_body(d_ref, b_ref, dinv_ref, binv_ref):
    ds_ = d_ref[0, :, 0:1] + d_ref[1, :, 0:1]
    bs_ = b_ref[0, :, 0:1] + b_ref[1, :, 0:1]
    dinv = jnp.where(ds_ > 0, 1.0 / ds_, 0.0)
    binv = jnp.where(bs_ > 0, 1.0 / bs_, 0.0)
    dinv_ref[...] = jnp.broadcast_to(dinv, (512, D))
    binv_ref[...] = jnp.broadcast_to(binv, (512, D))


def _tc_inv(dcnt, bcnt):
    return pl.pallas_call(
        _inv_body,
        grid=(NPAD // 512,),
        in_specs=[
            pl.BlockSpec((NC, 512, D), lambda i: (0, i, 0)),
            pl.BlockSpec((NC, 512, D), lambda i: (0, i, 0)),
        ],
        out_specs=[
            pl.BlockSpec((512, D), lambda i: (i, 0)),
            pl.BlockSpec((512, D), lambda i: (i, 0)),
        ],
        out_shape=[
            jax.ShapeDtypeStruct((NPAD, D), _f32),
            jax.ShapeDtypeStruct((NPAD, D), _f32),
        ],
    )(dcnt, bcnt)


def _mm_body(x_ref, w_ref, o_ref):
    o_ref[...] = jnp.dot(x_ref[...], w_ref[...],
                         preferred_element_type=_f32)


def _tc_mm(x, w):
    return pl.pallas_call(
        _mm_body,
        grid=(_GRID,),
        in_specs=[
            pl.BlockSpec((_RB, D), lambda i: (i, 0)),
            pl.BlockSpec((D, D), lambda i: (0, 0)),
        ],
        out_specs=pl.BlockSpec((_RB, D), lambda i: (i, 0)),
        out_shape=jax.ShapeDtypeStruct((N, D), _f32),
    )(x, w)


def _combine_body(p_ref, binv_ref, o_ref):
    o_ref[...] = binv_ref[...] * (p_ref[0] + p_ref[1])


def _tc_combine(part, binv_b):
    return pl.pallas_call(
        _combine_body,
        grid=(_GRID,),
        in_specs=[
            pl.BlockSpec((NC, _RB, D), lambda i: (0, i, 0)),
            pl.BlockSpec((_RB, D), lambda i: (i, 0)),
        ],
        out_specs=pl.BlockSpec((_RB, D), lambda i: (i, 0)),
        out_shape=jax.ShapeDtypeStruct((N, D), _f32),
    )(part, binv_b)


def _post_body(p_ref, dinv_ref, bias_ref, gamma_ref, beta_ref, w_ref, o_ref,
               *, use_ln):
    h = dinv_ref[...] * (p_ref[0] + p_ref[1]) + bias_ref[...]
    if use_ln:
        mu = jnp.mean(h, axis=-1, keepdims=True)
        var = jnp.mean((h - mu) ** 2, axis=-1, keepdims=True)
        h = (h - mu) / jnp.sqrt(var + 1e-5) * gamma_ref[...] + beta_ref[...]
    h = jnp.maximum(h, 0.0)
    o_ref[...] = jnp.dot(h, w_ref[...], preferred_element_type=_f32)


def _tc_post(part, dinv_b, bias, gamma, beta, w_next, use_ln):
    return pl.pallas_call(
        functools.partial(_post_body, use_ln=use_ln),
        grid=(_GRID,),
        in_specs=[
            pl.BlockSpec((NC, _RB, D), lambda i: (0, i, 0)),
            pl.BlockSpec((_RB, D), lambda i: (i, 0)),
            pl.BlockSpec((1, D), lambda i: (0, 0)),
            pl.BlockSpec((1, D), lambda i: (0, 0)),
            pl.BlockSpec((1, D), lambda i: (0, 0)),
            pl.BlockSpec((D, D), lambda i: (0, 0)),
        ],
        out_specs=pl.BlockSpec((_RB, D), lambda i: (i, 0)),
        out_shape=jax.ShapeDtypeStruct((N, D), _f32),
    )(part, dinv_b, bias, gamma, beta, w_next)


def _final_body(p_ref, dinv_ref, bias_ref, o_ref):
    o_ref[...] = dinv_ref[...] * (p_ref[0] + p_ref[1]) + bias_ref[...]


def _tc_final(part, dinv_b, bias):
    return pl.pallas_call(
        _final_body,
        grid=(_GRID,),
        in_specs=[
            pl.BlockSpec((NC, _RB, D), lambda i: (0, i, 0)),
            pl.BlockSpec((_RB, D), lambda i: (i, 0)),
            pl.BlockSpec((1, D), lambda i: (0, 0)),
        ],
        out_specs=pl.BlockSpec((_RB, D), lambda i: (i, 0)),
        out_shape=jax.ShapeDtypeStruct((N, D), _f32),
    )(part, dinv_b, bias)


def kernel(x, hyperedge_index, W1, b1, W2, b2, W3, b3, W4, b4, W5, b5,
           gamma, beta):
    nidx = hyperedge_index[0]
    eidx = hyperedge_index[1]
    zeros_big = jnp.zeros((NPAD, D), _f32)
    ones_rows = jnp.ones((C, D), _f32)

    def _pad_idx(idx, fill):
        a = idx.reshape(NT, EPT)
        a = jnp.pad(a, ((0, 0), (0, EPTP - EPT)), constant_values=fill)
        return a.reshape(-1)

    nidx_g = _pad_idx(nidx, 0)
    eidx_s = _pad_idx(eidx, DUMP)
    eidx_g = _pad_idx(eidx, 0)
    nidx_s = _pad_idx(nidx, DUMP)

    dcnt, bcnt = _sc_counts(nidx, eidx, ones_rows, zeros_big)
    dinv_b, binv_b = _tc_inv(dcnt, bcnt)

    Ws = [W1, W2, W3, W4, W5]
    bs = [b.reshape(1, D) for b in (b1, b2, b3, b4, b5)]
    gamma2 = gamma.reshape(1, D)
    beta2 = beta.reshape(1, D)

    xw = _tc_mm(x, W1)
    for i in range(5):
        pA = _sc_stage_pipe(xw, nidx_g, eidx_s, zeros_big)
        ef = _tc_combine(pA, binv_b)
        pB = _sc_stage_pipe(ef, eidx_g, nidx_s, zeros_big)
        if i < 4:
            xw = _tc_post(pB, dinv_b, bs[i], gamma2, beta2, Ws[i + 1],
                          use_ln=(i == 0))
        else:
            z = _tc_final(pB, dinv_b, bs[4])
    return z


# speedup vs baseline: 1.6428x; 1.6428x over previous
import functools

import jax
import jax.numpy as jnp
from jax import lax
from jax.experimental import pallas as pl
from jax.experimental.pallas import tpu as pltpu
from jax.experimental.pallas import tpu_sc as plsc

N = 10000
E = 320000
D = 128
NPAD = 10240
NC = 2
NS = 16
NT = NC * NS
EPT = E // NT
C = 200
CHUNKS = EPT // C
STRIPE = NPAD // NS

_mesh = plsc.VectorSubcoreMesh(core_axis_name="c", subcore_axis_name="s")

_f32 = jnp.float32


@functools.partial(
    pl.kernel,
    mesh=_mesh,
    out_type=[
        jax.ShapeDtypeStruct((NC, NPAD, D), _f32),
        jax.ShapeDtypeStruct((NC, NPAD, D), _f32),
    ],
    scratch_types=[
        pltpu.VMEM((C,), jnp.int32),
        pltpu.VMEM((C, D), _f32),
        pltpu.VMEM_SHARED((NPAD, D), _f32),
    ],
)
def _sc_counts(nidx_hbm, eidx_hbm, ones_hbm, zeros_hbm,
               dcnt_hbm, bcnt_hbm, idx_v, ones_v, acc_sh):
    c = lax.axis_index("c")
    s = lax.axis_index("s")
    pltpu.sync_copy(ones_hbm, ones_v)
    base = (c * NS + s) * EPT

    for idx_hbm, out_hbm in ((nidx_hbm, dcnt_hbm), (eidx_hbm, bcnt_hbm)):
        pltpu.sync_copy(zeros_hbm.at[pl.ds(s * STRIPE, STRIPE)],
                        acc_sh.at[pl.ds(s * STRIPE, STRIPE)])
        plsc.subcore_barrier()

        @pl.loop(0, CHUNKS)
        def _(j):
            off = base + j * C
            pltpu.sync_copy(idx_hbm.at[pl.ds(off, C)], idx_v)
            pltpu.sync_copy(ones_v, acc_sh.at[idx_v], add=True)

        plsc.subcore_barrier()
        pltpu.sync_copy(acc_sh.at[pl.ds(s * STRIPE, STRIPE)],
                        out_hbm.at[c, pl.ds(s * STRIPE, STRIPE)])
        plsc.subcore_barrier()


CP = 320
CHP = 32
EPTP = CP * CHP
EPAD = NT * EPTP
DUMP = NPAD - 1


@functools.partial(
    pl.kernel,
    mesh=_mesh,
    out_type=jax.ShapeDtypeStruct((NC, NPAD, D), _f32),
    scratch_types=[
        pltpu.VMEM((CP,), jnp.int32),
        pltpu.VMEM((CP,), jnp.int32),
        pltpu.VMEM((CP, D), _f32),
        pltpu.VMEM_SHARED((NPAD, D), _f32),
        pltpu.SemaphoreType.DMA,
    ],
)
def _sc_stage_pipe(feat_hbm, gidx_hbm, sidx_hbm, zeros_hbm, out_hbm,
                   gi_v, si_v, rows_v, acc_sh, sem):
    c = lax.axis_index("c")
    s = lax.axis_index("s")
    pltpu.sync_copy(zeros_hbm.at[pl.ds(s * STRIPE, STRIPE)],
                    acc_sh.at[pl.ds(s * STRIPE, STRIPE)])
    plsc.subcore_barrier()
    base = (c * NS + s) * EPTP

    @pl.loop(0, CHP)
    def _(j):
        off = base + j * CP
        pltpu.sync_copy(gidx_hbm.at[pl.ds(off, CP)], gi_v)
        pltpu.sync_copy(sidx_hbm.at[pl.ds(off, CP)], si_v)
        pltpu.async_copy(feat_hbm.at[gi_v], rows_v, sem).wait()
        pltpu.sync_copy(rows_v, acc_sh.at[si_v], add=True)

    plsc.subcore_barrier()
    pltpu.sync_copy(acc_sh.at[pl.ds(s * STRIPE, STRIPE)],
                    out_hbm.at[c, pl.ds(s * STRIPE, STRIPE)])


_RB = 400
_GRID = N // _RB


def _inv_body(d_ref, b_ref, dinv_ref, binv_ref):
    ds_ = d_ref[0, :, 0:1] + d_ref[1, :, 0:1]
    bs_ = b_ref[0, :, 0:1] + b_ref[1, :, 0:1]
    dinv = jnp.where(ds_ > 0, 1.0 / ds_, 0.0)
    binv = jnp.where(bs_ > 0, 1.0 / bs_, 0.0)
    dinv_ref[...] = jnp.broadcast_to(dinv, (512, D))
    binv_ref[...] = jnp.broadcast_to(binv, (512, D))


def _tc_inv(dcnt, bcnt):
    return pl.pallas_call(
        _inv_body,
        grid=(NPAD // 512,),
        in_specs=[
            pl.BlockSpec((NC, 512, D), lambda i: (0, i, 0)),
            pl.BlockSpec((NC, 512, D), lambda i: (0, i, 0)),
        ],
        out_specs=[
            pl.BlockSpec((512, D), lambda i: (i, 0)),
            pl.BlockSpec((512, D), lambda i: (i, 0)),
        ],
        out_shape=[
            jax.ShapeDtypeStruct((NPAD, D), _f32),
            jax.ShapeDtypeStruct((NPAD, D), _f32),
        ],
    )(dcnt, bcnt)


def _mm_body(x_ref, w_ref, o_ref):
    o_ref[...] = jnp.dot(x_ref[...], w_ref[...],
                         preferred_element_type=_f32)


def _tc_mm(x, w):
    return pl.pallas_call(
        _mm_body,
        grid=(_GRID,),
        in_specs=[
            pl.BlockSpec((_RB, D), lambda i: (i, 0)),
            pl.BlockSpec((D, D), lambda i: (0, 0)),
        ],
        out_specs=pl.BlockSpec((_RB, D), lambda i: (i, 0)),
        out_shape=jax.ShapeDtypeStruct((N, D), _f32),
    )(x, w)


def _combine_body(p_ref, binv_ref, o_ref):
    o_ref[...] = binv_ref[...] * (p_ref[0] + p_ref[1])


def _tc_combine(part, binv_b):
    return pl.pallas_call(
        _combine_body,
        grid=(_GRID,),
        in_specs=[
            pl.BlockSpec((NC, _RB, D), lambda i: (0, i, 0)),
            pl.BlockSpec((_RB, D), lambda i: (i, 0)),
        ],
        out_specs=pl.BlockSpec((_RB, D), lambda i: (i, 0)),
        out_shape=jax.ShapeDtypeStruct((N, D), _f32),
    )(part, binv_b)


def _post_body(p_ref, dinv_ref, bias_ref, gamma_ref, beta_ref, w_ref, o_ref,
               *, use_ln):
    h = dinv_ref[...] * (p_ref[0] + p_ref[1]) + bias_ref[...]
    if use_ln:
        mu = jnp.mean(h, axis=-1, keepdims=True)
        var = jnp.mean((h - mu) ** 2, axis=-1, keepdims=True)
        h = (h - mu) / jnp.sqrt(var + 1e-5) * gamma_ref[...] + beta_ref[...]
    h = jnp.maximum(h, 0.0)
    o_ref[...] = jnp.dot(h, w_ref[...], preferred_element_type=_f32)


def _tc_post(part, dinv_b, bias, gamma, beta, w_next, use_ln):
    return pl.pallas_call(
        functools.partial(_post_body, use_ln=use_ln),
        grid=(_GRID,),
        in_specs=[
            pl.BlockSpec((NC, _RB, D), lambda i: (0, i, 0)),
            pl.BlockSpec((_RB, D), lambda i: (i, 0)),
            pl.BlockSpec((1, D), lambda i: (0, 0)),
            pl.BlockSpec((1, D), lambda i: (0, 0)),
            pl.BlockSpec((1, D), lambda i: (0, 0)),
            pl.BlockSpec((D, D), lambda i: (0, 0)),
        ],
        out_specs=pl.BlockSpec((_RB, D), lambda i: (i, 0)),
        out_shape=jax.ShapeDtypeStruct((N, D), _f32),
    )(part, dinv_b, bias, gamma, beta, w_next)


def _final_body(p_ref, dinv_ref, bias_ref, o_ref):
    o_ref[...] = dinv_ref[...] * (p_ref[0] + p_ref[1]) + bias_ref[...]


def _tc_final(part, dinv_b, bias):
    return pl.pallas_call(
        _final_body,
        grid=(_GRID,),
        in_specs=[
            pl.BlockSpec((NC, _RB, D), lambda i: (0, i, 0)),
            pl.BlockSpec((_RB, D), lambda i: (i, 0)),
            pl.BlockSpec((1, D), lambda i: (0, 0)),
        ],
        out_specs=pl.BlockSpec((_RB, D), lambda i: (i, 0)),
        out_shape=jax.ShapeDtypeStruct((N, D), _f32),
    )(part, dinv_b, bias)


def kernel(x, hyperedge_index, W1, b1, W2, b2, W3, b3, W4, b4, W5, b5,
           gamma, beta):
    nidx = hyperedge_index[0]
    eidx = hyperedge_index[1]
    zeros_big = jnp.zeros((NPAD, D), _f32)
    ones_rows = jnp.ones((C, D), _f32)

    def _pad_idx(idx, fill):
        a = idx.reshape(NT, EPT)
        a = jnp.pad(a, ((0, 0), (0, EPTP - EPT)), constant_values=fill)
        return a.reshape(-1)

    nidx_g = _pad_idx(nidx, 0)
    eidx_s = _pad_idx(eidx, DUMP)
    eidx_g = _pad_idx(eidx, 0)
    nidx_s = _pad_idx(nidx, DUMP)

    dcnt, bcnt = _sc_counts(nidx, eidx, ones_rows, zeros_big)
    dinv_b, binv_b = _tc_inv(dcnt, bcnt)

    Ws = [W1, W2, W3, W4, W5]
    bs = [b.reshape(1, D) for b in (b1, b2, b3, b4, b5)]
    gamma2 = gamma.reshape(1, D)
    beta2 = beta.reshape(1, D)

    xw = _tc_mm(x, W1)
    for i in range(5):
        pA = _sc_stage_pipe(xw, nidx_g, eidx_s, zeros_big)
        ef = _tc_combine(pA, binv_b)
        pB = _sc_stage_pipe(ef, eidx_g, nidx_s, zeros_big)
        if i < 4:
            xw = _tc_post(pB, dinv_b, bs[i], gamma2, beta2, Ws[i + 1],
                          use_ln=(i == 0))
        else:
            z = _tc_final(pB, dinv_b, bs[4])
    return z


# speedup vs baseline: 2.0087x; 1.2227x over previous
import functools

import jax
import jax.numpy as jnp
from jax import lax
from jax.experimental import pallas as pl
from jax.experimental.pallas import tpu as pltpu
from jax.experimental.pallas import tpu_sc as plsc

N = 10000
E = 320000
D = 128
NPAD = 10240
NC = 2
NS = 16
NT = NC * NS
EPT = E // NT
C = 200
CHUNKS = EPT // C
STRIPE = NPAD // NS

_mesh = plsc.VectorSubcoreMesh(core_axis_name="c", subcore_axis_name="s")

_f32 = jnp.float32


@functools.partial(
    pl.kernel,
    mesh=_mesh,
    out_type=[
        jax.ShapeDtypeStruct((NC, NPAD, D), _f32),
        jax.ShapeDtypeStruct((NC, NPAD, D), _f32),
    ],
    scratch_types=[
        pltpu.VMEM((C,), jnp.int32),
        pltpu.VMEM((C, D), _f32),
        pltpu.VMEM_SHARED((NPAD, D), _f32),
    ],
)
def _sc_counts(nidx_hbm, eidx_hbm, ones_hbm, zeros_hbm,
               dcnt_hbm, bcnt_hbm, idx_v, ones_v, acc_sh):
    c = lax.axis_index("c")
    s = lax.axis_index("s")
    pltpu.sync_copy(ones_hbm, ones_v)
    base = (c * NS + s) * EPT

    for idx_hbm, out_hbm in ((nidx_hbm, dcnt_hbm), (eidx_hbm, bcnt_hbm)):
        pltpu.sync_copy(zeros_hbm.at[pl.ds(s * STRIPE, STRIPE)],
                        acc_sh.at[pl.ds(s * STRIPE, STRIPE)])
        plsc.subcore_barrier()

        @pl.loop(0, CHUNKS)
        def _(j):
            off = base + j * C
            pltpu.sync_copy(idx_hbm.at[pl.ds(off, C)], idx_v)
            pltpu.sync_copy(ones_v, acc_sh.at[idx_v], add=True)

        plsc.subcore_barrier()
        pltpu.sync_copy(acc_sh.at[pl.ds(s * STRIPE, STRIPE)],
                        out_hbm.at[c, pl.ds(s * STRIPE, STRIPE)])
        plsc.subcore_barrier()


CP = 128
CHP = 79
EPTP = CP * CHP
EPAD = NT * EPTP
DUMP = NPAD - 1


@functools.partial(
    pl.kernel,
    mesh=_mesh,
    out_type=jax.ShapeDtypeStruct((NC, NPAD, D), _f32),
    scratch_types=[
        pltpu.VMEM((CP,), jnp.int32),
        pltpu.VMEM((CP,), jnp.int32),
        pltpu.VMEM((CP, D), _f32),
        pltpu.VMEM_SHARED((NPAD, D), _f32),
        pltpu.SemaphoreType.DMA,
    ],
)
def _sc_stage_pipe(feat_hbm, gidx_hbm, sidx_hbm, zeros_hbm, out_hbm,
                   gi_v, si_v, rows_v, acc_sh, sem):
    c = lax.axis_index("c")
    s = lax.axis_index("s")
    pltpu.sync_copy(zeros_hbm.at[pl.ds(s * STRIPE, STRIPE)],
                    acc_sh.at[pl.ds(s * STRIPE, STRIPE)])
    plsc.subcore_barrier()
    base = (c * NS + s) * EPTP

    @pl.loop(0, CHP)
    def _(j):
        off = base + j * CP
        pltpu.sync_copy(gidx_hbm.at[pl.ds(off, CP)], gi_v)
        pltpu.sync_copy(sidx_hbm.at[pl.ds(off, CP)], si_v)
        pltpu.async_copy(feat_hbm.at[gi_v], rows_v, sem).wait()
        pltpu.sync_copy(rows_v, acc_sh.at[si_v], add=True)

    plsc.subcore_barrier()
    pltpu.sync_copy(acc_sh.at[pl.ds(s * STRIPE, STRIPE)],
                    out_hbm.at[c, pl.ds(s * STRIPE, STRIPE)])


_RB = 400
_GRID = N // _RB


def _inv_body(d_ref, b_ref, dinv_ref, binv_ref):
    ds_ = d_ref[0, :, 0:1] + d_ref[1, :, 0:1]
    bs_ = b_ref[0, :, 0:1] + b_ref[1, :, 0:1]
    dinv = jnp.where(ds_ > 0, 1.0 / ds_, 0.0)
    binv = jnp.where(bs_ > 0, 1.0 / bs_, 0.0)
    dinv_ref[...] = jnp.broadcast_to(dinv, (512, D))
    binv_ref[...] = jnp.broadcast_to(binv, (512, D))


def _tc_inv(dcnt, bcnt):
    return pl.pallas_call(
        _inv_body,
        grid=(NPAD // 512,),
        in_specs=[
            pl.BlockSpec((NC, 512, D), lambda i: (0, i, 0)),
            pl.BlockSpec((NC, 512, D), lambda i: (0, i, 0)),
        ],
        out_specs=[
            pl.BlockSpec((512, D), lambda i: (i, 0)),
            pl.BlockSpec((512, D), lambda i: (i, 0)),
        ],
        out_shape=[
            jax.ShapeDtypeStruct((NPAD, D), _f32),
            jax.ShapeDtypeStruct((NPAD, D), _f32),
        ],
    )(dcnt, bcnt)


def _mm_body(x_ref, w_ref, o_ref):
    o_ref[...] = jnp.dot(x_ref[...], w_ref[...],
                         preferred_element_type=_f32)


def _tc_mm(x, w):
    return pl.pallas_call(
        _mm_body,
        grid=(_GRID,),
        in_specs=[
            pl.BlockSpec((_RB, D), lambda i: (i, 0)),
            pl.BlockSpec((D, D), lambda i: (0, 0)),
        ],
        out_specs=pl.BlockSpec((_RB, D), lambda i: (i, 0)),
        out_shape=jax.ShapeDtypeStruct((N, D), _f32),
    )(x, w)


def _combine_body(p_ref, binv_ref, o_ref):
    o_ref[...] = binv_ref[...] * (p_ref[0] + p_ref[1])


def _tc_combine(part, binv_b):
    return pl.pallas_call(
        _combine_body,
        grid=(_GRID,),
        in_specs=[
            pl.BlockSpec((NC, _RB, D), lambda i: (0, i, 0)),
            pl.BlockSpec((_RB, D), lambda i: (i, 0)),
        ],
        out_specs=pl.BlockSpec((_RB, D), lambda i: (i, 0)),
        out_shape=jax.ShapeDtypeStruct((N, D), _f32),
    )(part, binv_b)


def _post_body(p_ref, dinv_ref, bias_ref, gamma_ref, beta_ref, w_ref, o_ref,
               *, use_ln):
    h = dinv_ref[...] * (p_ref[0] + p_ref[1]) + bias_ref[...]
    if use_ln:
        mu = jnp.mean(h, axis=-1, keepdims=True)
        var = jnp.mean((h - mu) ** 2, axis=-1, keepdims=True)
        h = (h - mu) / jnp.sqrt(var + 1e-5) * gamma_ref[...] + beta_ref[...]
    h = jnp.maximum(h, 0.0)
    o_ref[...] = jnp.dot(h, w_ref[...], preferred_element_type=_f32)


def _tc_post(part, dinv_b, bias, gamma, beta, w_next, use_ln):
    return pl.pallas_call(
        functools.partial(_post_body, use_ln=use_ln),
        grid=(_GRID,),
        in_specs=[
            pl.BlockSpec((NC, _RB, D), lambda i: (0, i, 0)),
            pl.BlockSpec((_RB, D), lambda i: (i, 0)),
            pl.BlockSpec((1, D), lambda i: (0, 0)),
            pl.BlockSpec((1, D), lambda i: (0, 0)),
            pl.BlockSpec((1, D), lambda i: (0, 0)),
            pl.BlockSpec((D, D), lambda i: (0, 0)),
        ],
        out_specs=pl.BlockSpec((_RB, D), lambda i: (i, 0)),
        out_shape=jax.ShapeDtypeStruct((N, D), _f32),
    )(part, dinv_b, bias, gamma, beta, w_next)


def _final_body(p_ref, dinv_ref, bias_ref, o_ref):
    o_ref[...] = dinv_ref[...] * (p_ref[0] + p_ref[1]) + bias_ref[...]


def _tc_final(part, dinv_b, bias):
    return pl.pallas_call(
        _final_body,
        grid=(_GRID,),
        in_specs=[
            pl.BlockSpec((NC, _RB, D), lambda i: (0, i, 0)),
            pl.BlockSpec((_RB, D), lambda i: (i, 0)),
            pl.BlockSpec((1, D), lambda i: (0, 0)),
        ],
        out_specs=pl.BlockSpec((_RB, D), lambda i: (i, 0)),
        out_shape=jax.ShapeDtypeStruct((N, D), _f32),
    )(part, dinv_b, bias)


def kernel(x, hyperedge_index, W1, b1, W2, b2, W3, b3, W4, b4, W5, b5,
           gamma, beta):
    nidx = hyperedge_index[0]
    eidx = hyperedge_index[1]
    zeros_big = jnp.zeros((NPAD, D), _f32)
    ones_rows = jnp.ones((C, D), _f32)

    def _pad_idx(idx, fill):
        a = idx.reshape(NT, EPT)
        a = jnp.pad(a, ((0, 0), (0, EPTP - EPT)), constant_values=fill)
        return a.reshape(-1)

    nidx_g = _pad_idx(nidx, 0)
    eidx_s = _pad_idx(eidx, DUMP)
    eidx_g = _pad_idx(eidx, 0)
    nidx_s = _pad_idx(nidx, DUMP)

    dcnt, bcnt = _sc_counts(nidx, eidx, ones_rows, zeros_big)
    dinv_b, binv_b = _tc_inv(dcnt, bcnt)

    Ws = [W1, W2, W3, W4, W5]
    bs = [b.reshape(1, D) for b in (b1, b2, b3, b4, b5)]
    gamma2 = gamma.reshape(1, D)
    beta2 = beta.reshape(1, D)

    xw = _tc_mm(x, W1)
    for i in range(5):
        pA = _sc_stage_pipe(xw, nidx_g, eidx_s, zeros_big)
        ef = _tc_combine(pA, binv_b)
        pB = _sc_stage_pipe(ef, eidx_g, nidx_s, zeros_big)
        if i < 4:
            xw = _tc_post(pB, dinv_b, bs[i], gamma2, beta2, Ws[i + 1],
                          use_ln=(i == 0))
        else:
            z = _tc_final(pB, dinv_b, bs[4])
    return z


# speedup vs baseline: 2.8600x; 1.4238x over previous
import functools

import jax
import jax.numpy as jnp
from jax import lax
from jax.experimental import pallas as pl
from jax.experimental.pallas import tpu as pltpu
from jax.experimental.pallas import tpu_sc as plsc

N = 10000
E = 320000
D = 128
NPAD = 10240
NC = 2
NS = 16
NT = NC * NS
EPT = E // NT
C = 200
CHUNKS = EPT // C
STRIPE = NPAD // NS

_mesh = plsc.VectorSubcoreMesh(core_axis_name="c", subcore_axis_name="s")

_f32 = jnp.float32


@functools.partial(
    pl.kernel,
    mesh=_mesh,
    out_type=[
        jax.ShapeDtypeStruct((NC, NPAD, D), _f32),
        jax.ShapeDtypeStruct((NC, NPAD, D), _f32),
    ],
    scratch_types=[
        pltpu.VMEM((C,), jnp.int32),
        pltpu.VMEM((C, D), _f32),
        pltpu.VMEM_SHARED((NPAD, D), _f32),
    ],
)
def _sc_counts(nidx_hbm, eidx_hbm, ones_hbm, zeros_hbm,
               dcnt_hbm, bcnt_hbm, idx_v, ones_v, acc_sh):
    c = lax.axis_index("c")
    s = lax.axis_index("s")
    pltpu.sync_copy(ones_hbm, ones_v)
    base = (c * NS + s) * EPT

    for idx_hbm, out_hbm in ((nidx_hbm, dcnt_hbm), (eidx_hbm, bcnt_hbm)):
        pltpu.sync_copy(zeros_hbm.at[pl.ds(s * STRIPE, STRIPE)],
                        acc_sh.at[pl.ds(s * STRIPE, STRIPE)])
        plsc.subcore_barrier()

        @pl.loop(0, CHUNKS)
        def _(j):
            off = base + j * C
            pltpu.sync_copy(idx_hbm.at[pl.ds(off, C)], idx_v)
            pltpu.sync_copy(ones_v, acc_sh.at[idx_v], add=True)

        plsc.subcore_barrier()
        pltpu.sync_copy(acc_sh.at[pl.ds(s * STRIPE, STRIPE)],
                        out_hbm.at[c, pl.ds(s * STRIPE, STRIPE)])
        plsc.subcore_barrier()


CP = 128
CHP = 79
EPTP = CP * CHP
EPAD = NT * EPTP
DUMP = NPAD - 1


@functools.partial(
    pl.kernel,
    mesh=_mesh,
    out_type=jax.ShapeDtypeStruct((NC, NPAD, D), _f32),
    scratch_types=[
        pltpu.VMEM((CP,), jnp.int32),
        pltpu.VMEM((CP,), jnp.int32),
        pltpu.VMEM((CP, D), _f32),
        pltpu.VMEM_SHARED((NPAD, D), _f32),
        pltpu.SemaphoreType.DMA,
    ],
)
def _sc_stage_pipe(feat_hbm, gidx_hbm, sidx_hbm, zeros_hbm, out_hbm,
                   gi_v, si_v, rows_v, acc_sh, sem):
    c = lax.axis_index("c")
    s = lax.axis_index("s")
    pltpu.sync_copy(zeros_hbm.at[pl.ds(s * STRIPE, STRIPE)],
                    acc_sh.at[pl.ds(s * STRIPE, STRIPE)])
    plsc.subcore_barrier()
    base = (c * NS + s) * EPTP

    @pl.loop(0, CHP)
    def _(j):
        off = base + j * CP
        pltpu.sync_copy(gidx_hbm.at[pl.ds(off, CP)], gi_v)
        pltpu.sync_copy(sidx_hbm.at[pl.ds(off, CP)], si_v)
        pltpu.async_copy(feat_hbm.at[gi_v], rows_v, sem).wait()
        pltpu.sync_copy(rows_v, acc_sh.at[si_v], add=True)

    plsc.subcore_barrier()
    pltpu.sync_copy(acc_sh.at[pl.ds(s * STRIPE, STRIPE)],
                    out_hbm.at[c, pl.ds(s * STRIPE, STRIPE)])


_RB = 400
_GRID = N // _RB


def _inv_body(d_ref, b_ref, dinv_ref, binv_ref):
    ds_ = d_ref[0, :, 0:1] + d_ref[1, :, 0:1]
    bs_ = b_ref[0, :, 0:1] + b_ref[1, :, 0:1]
    dinv = jnp.where(ds_ > 0, 1.0 / ds_, 0.0)
    binv = jnp.where(bs_ > 0, 1.0 / bs_, 0.0)
    dinv_ref[...] = jnp.broadcast_to(dinv, (512, D))
    binv_ref[...] = jnp.broadcast_to(binv, (512, D))


def _tc_inv(dcnt, bcnt):
    return pl.pallas_call(
        _inv_body,
        grid=(NPAD // 512,),
        in_specs=[
            pl.BlockSpec((NC, 512, D), lambda i: (0, i, 0)),
            pl.BlockSpec((NC, 512, D), lambda i: (0, i, 0)),
        ],
        out_specs=[
            pl.BlockSpec((512, D), lambda i: (i, 0)),
            pl.BlockSpec((512, D), lambda i: (i, 0)),
        ],
        out_shape=[
            jax.ShapeDtypeStruct((NPAD, D), _f32),
            jax.ShapeDtypeStruct((NPAD, D), _f32),
        ],
    )(dcnt, bcnt)


def _mm_body(x_ref, w_ref, o_ref):
    o_ref[...] = jnp.dot(x_ref[...], w_ref[...],
                         preferred_element_type=_f32)


def _tc_mm(x, w):
    return pl.pallas_call(
        _mm_body,
        grid=(_GRID,),
        in_specs=[
            pl.BlockSpec((_RB, D), lambda i: (i, 0)),
            pl.BlockSpec((D, D), lambda i: (0, 0)),
        ],
        out_specs=pl.BlockSpec((_RB, D), lambda i: (i, 0)),
        out_shape=jax.ShapeDtypeStruct((N, D), _f32),
    )(x, w)


def _combine_body(p_ref, binv_ref, o_ref):
    o_ref[...] = binv_ref[...] * (p_ref[0] + p_ref[1])


def _tc_combine(part, binv_b):
    return pl.pallas_call(
        _combine_body,
        grid=(_GRID,),
        in_specs=[
            pl.BlockSpec((NC, _RB, D), lambda i: (0, i, 0)),
            pl.BlockSpec((_RB, D), lambda i: (i, 0)),
        ],
        out_specs=pl.BlockSpec((_RB, D), lambda i: (i, 0)),
        out_shape=jax.ShapeDtypeStruct((N, D), _f32),
    )(part, binv_b)


def _post_body(p_ref, dinv_ref, bias_ref, gamma_ref, beta_ref, w_ref, o_ref,
               *, use_ln):
    h = dinv_ref[...] * (p_ref[0] + p_ref[1]) + bias_ref[...]
    if use_ln:
        mu = jnp.mean(h, axis=-1, keepdims=True)
        var = jnp.mean((h - mu) ** 2, axis=-1, keepdims=True)
        h = (h - mu) / jnp.sqrt(var + 1e-5) * gamma_ref[...] + beta_ref[...]
    h = jnp.maximum(h, 0.0)
    o_ref[...] = jnp.dot(h, w_ref[...], preferred_element_type=_f32)


def _tc_post(part, dinv_b, bias, gamma, beta, w_next, use_ln):
    return pl.pallas_call(
        functools.partial(_post_body, use_ln=use_ln),
        grid=(_GRID,),
        in_specs=[
            pl.BlockSpec((NC, _RB, D), lambda i: (0, i, 0)),
            pl.BlockSpec((_RB, D), lambda i: (i, 0)),
            pl.BlockSpec((1, D), lambda i: (0, 0)),
            pl.BlockSpec((1, D), lambda i: (0, 0)),
            pl.BlockSpec((1, D), lambda i: (0, 0)),
            pl.BlockSpec((D, D), lambda i: (0, 0)),
        ],
        out_specs=pl.BlockSpec((_RB, D), lambda i: (i, 0)),
        out_shape=jax.ShapeDtypeStruct((N, D), _f32),
    )(part, dinv_b, bias, gamma, beta, w_next)


def _final_body(p_ref, dinv_ref, bias_ref, o_ref):
    o_ref[...] = dinv_ref[...] * (p_ref[0] + p_ref[1]) + bias_ref[...]


def _tc_final(part, dinv_b, bias):
    return pl.pallas_call(
        _final_body,
        grid=(_GRID,),
        in_specs=[
            pl.BlockSpec((NC, _RB, D), lambda i: (0, i, 0)),
            pl.BlockSpec((_RB, D), lambda i: (i, 0)),
            pl.BlockSpec((1, D), lambda i: (0, 0)),
        ],
        out_specs=pl.BlockSpec((_RB, D), lambda i: (i, 0)),
        out_shape=jax.ShapeDtypeStruct((N, D), _f32),
    )(part, dinv_b, bias)


def kernel(x, hyperedge_index, W1, b1, W2, b2, W3, b3, W4, b4, W5, b5,
           gamma, beta):
    nidx = hyperedge_index[0]
    eidx = hyperedge_index[1]
    zeros_big = jnp.zeros((NPAD, D), _f32)
    ones_rows = jnp.ones((C, D), _f32)

    padn = EPTP - EPT
    t_ = jnp.arange(NT, dtype=jnp.int32)[:, None]
    j_ = jnp.arange(padn, dtype=jnp.int32)[None, :]
    gfill = (t_ * 613 + j_ * 97) % N
    sfill = N + (t_ * 7 + j_) % (NPAD - N)

    def _pad_idx(idx, fill):
        a = idx.reshape(NT, EPT)
        return jnp.concatenate([a, fill], axis=1).reshape(-1)

    nidx_g = _pad_idx(nidx, gfill)
    eidx_s = _pad_idx(eidx, sfill)
    eidx_g = _pad_idx(eidx, gfill)
    nidx_s = _pad_idx(nidx, sfill)

    dcnt, bcnt = _sc_counts(nidx, eidx, ones_rows, zeros_big)
    dinv_b, binv_b = _tc_inv(dcnt, bcnt)

    Ws = [W1, W2, W3, W4, W5]
    bs = [b.reshape(1, D) for b in (b1, b2, b3, b4, b5)]
    gamma2 = gamma.reshape(1, D)
    beta2 = beta.reshape(1, D)

    xw = _tc_mm(x, W1)
    for i in range(5):
        pA = _sc_stage_pipe(xw, nidx_g, eidx_s, zeros_big)
        ef = _tc_combine(pA, binv_b)
        pB = _sc_stage_pipe(ef, eidx_g, nidx_s, zeros_big)
        if i < 4:
            xw = _tc_post(pB, dinv_b, bs[i], gamma2, beta2, Ws[i + 1],
                          use_ln=(i == 0))
        else:
            z = _tc_final(pB, dinv_b, bs[4])
    return z


# speedup vs baseline: 3.7819x; 1.3223x over previous
import functools

import jax
import jax.numpy as jnp
from jax import lax
from jax.experimental import pallas as pl
from jax.experimental.pallas import tpu as pltpu
from jax.experimental.pallas import tpu_sc as plsc

N = 10000
E = 320000
D = 128
NPAD = 10240
NC = 2
NS = 16
NT = NC * NS
EPT = E // NT
C = 200
CHUNKS = EPT // C
STRIPE = NPAD // NS

_mesh = plsc.VectorSubcoreMesh(core_axis_name="c", subcore_axis_name="s")

_f32 = jnp.float32


@functools.partial(
    pl.kernel,
    mesh=_mesh,
    out_type=[
        jax.ShapeDtypeStruct((NC, NPAD, D), _f32),
        jax.ShapeDtypeStruct((NC, NPAD, D), _f32),
    ],
    scratch_types=[
        pltpu.VMEM((C,), jnp.int32),
        pltpu.VMEM((C, D), _f32),
        pltpu.VMEM_SHARED((NPAD, D), _f32),
    ],
)
def _sc_counts(nidx_hbm, eidx_hbm, ones_hbm, zeros_hbm,
               dcnt_hbm, bcnt_hbm, idx_v, ones_v, acc_sh):
    c = lax.axis_index("c")
    s = lax.axis_index("s")
    pltpu.sync_copy(ones_hbm, ones_v)
    base = (c * NS + s) * EPT

    for idx_hbm, out_hbm in ((nidx_hbm, dcnt_hbm), (eidx_hbm, bcnt_hbm)):
        pltpu.sync_copy(zeros_hbm.at[pl.ds(s * STRIPE, STRIPE)],
                        acc_sh.at[pl.ds(s * STRIPE, STRIPE)])
        plsc.subcore_barrier()

        @pl.loop(0, CHUNKS)
        def _(j):
            off = base + j * C
            pltpu.sync_copy(idx_hbm.at[pl.ds(off, C)], idx_v)
            pltpu.sync_copy(ones_v, acc_sh.at[idx_v], add=True)

        plsc.subcore_barrier()
        pltpu.sync_copy(acc_sh.at[pl.ds(s * STRIPE, STRIPE)],
                        out_hbm.at[c, pl.ds(s * STRIPE, STRIPE)])
        plsc.subcore_barrier()


CP = 320
CHP = 32
EPTP = CP * CHP
EPAD = NT * EPTP
DUMP = NPAD - 1


@functools.partial(
    pl.kernel,
    mesh=_mesh,
    out_type=jax.ShapeDtypeStruct((NC, NPAD, D), _f32),
    scratch_types=[
        pltpu.VMEM((CP,), jnp.int32),
        pltpu.VMEM((CP,), jnp.int32),
        pltpu.VMEM((CP, D), _f32),
        pltpu.VMEM_SHARED((NPAD, D), _f32),
        pltpu.SemaphoreType.DMA,
    ],
)
def _sc_stage_pipe(feat_hbm, gidx_hbm, sidx_hbm, zeros_hbm, out_hbm,
                   gi_v, si_v, rows_v, acc_sh, sem):
    c = lax.axis_index("c")
    s = lax.axis_index("s")
    pltpu.sync_copy(zeros_hbm.at[pl.ds(s * STRIPE, STRIPE)],
                    acc_sh.at[pl.ds(s * STRIPE, STRIPE)])
    plsc.subcore_barrier()
    base = (c * NS + s) * EPTP

    @pl.loop(0, CHP)
    def _(j):
        off = base + j * CP
        pltpu.sync_copy(gidx_hbm.at[pl.ds(off, CP)], gi_v)
        pltpu.sync_copy(sidx_hbm.at[pl.ds(off, CP)], si_v)
        pltpu.async_copy(feat_hbm.at[gi_v], rows_v, sem).wait()
        pltpu.sync_copy(rows_v, acc_sh.at[si_v], add=True)

    plsc.subcore_barrier()
    pltpu.sync_copy(acc_sh.at[pl.ds(s * STRIPE, STRIPE)],
                    out_hbm.at[c, pl.ds(s * STRIPE, STRIPE)])


_RB = 400
_GRID = N // _RB


def _inv_body(d_ref, b_ref, dinv_ref, binv_ref):
    ds_ = d_ref[0, :, 0:1] + d_ref[1, :, 0:1]
    bs_ = b_ref[0, :, 0:1] + b_ref[1, :, 0:1]
    dinv = jnp.where(ds_ > 0, 1.0 / ds_, 0.0)
    binv = jnp.where(bs_ > 0, 1.0 / bs_, 0.0)
    dinv_ref[...] = jnp.broadcast_to(dinv, (512, D))
    binv_ref[...] = jnp.broadcast_to(binv, (512, D))


def _tc_inv(dcnt, bcnt):
    return pl.pallas_call(
        _inv_body,
        grid=(NPAD // 512,),
        in_specs=[
            pl.BlockSpec((NC, 512, D), lambda i: (0, i, 0)),
            pl.BlockSpec((NC, 512, D), lambda i: (0, i, 0)),
        ],
        out_specs=[
            pl.BlockSpec((512, D), lambda i: (i, 0)),
            pl.BlockSpec((512, D), lambda i: (i, 0)),
        ],
        out_shape=[
            jax.ShapeDtypeStruct((NPAD, D), _f32),
            jax.ShapeDtypeStruct((NPAD, D), _f32),
        ],
    )(dcnt, bcnt)


def _mm_body(x_ref, w_ref, o_ref):
    o_ref[...] = jnp.dot(x_ref[...], w_ref[...],
                         preferred_element_type=_f32)


def _tc_mm(x, w):
    return pl.pallas_call(
        _mm_body,
        grid=(_GRID,),
        in_specs=[
            pl.BlockSpec((_RB, D), lambda i: (i, 0)),
            pl.BlockSpec((D, D), lambda i: (0, 0)),
        ],
        out_specs=pl.BlockSpec((_RB, D), lambda i: (i, 0)),
        out_shape=jax.ShapeDtypeStruct((N, D), _f32),
    )(x, w)


def _combine_body(p_ref, binv_ref, o_ref):
    o_ref[...] = binv_ref[...] * (p_ref[0] + p_ref[1])


def _tc_combine(part, binv_b):
    return pl.pallas_call(
        _combine_body,
        grid=(_GRID,),
        in_specs=[
            pl.BlockSpec((NC, _RB, D), lambda i: (0, i, 0)),
            pl.BlockSpec((_RB, D), lambda i: (i, 0)),
        ],
        out_specs=pl.BlockSpec((_RB, D), lambda i: (i, 0)),
        out_shape=jax.ShapeDtypeStruct((N, D), _f32),
    )(part, binv_b)


def _post_body(p_ref, dinv_ref, bias_ref, gamma_ref, beta_ref, w_ref, o_ref,
               *, use_ln):
    h = dinv_ref[...] * (p_ref[0] + p_ref[1]) + bias_ref[...]
    if use_ln:
        mu = jnp.mean(h, axis=-1, keepdims=True)
        var = jnp.mean((h - mu) ** 2, axis=-1, keepdims=True)
        h = (h - mu) / jnp.sqrt(var + 1e-5) * gamma_ref[...] + beta_ref[...]
    h = jnp.maximum(h, 0.0)
    o_ref[...] = jnp.dot(h, w_ref[...], preferred_element_type=_f32)


def _tc_post(part, dinv_b, bias, gamma, beta, w_next, use_ln):
    return pl.pallas_call(
        functools.partial(_post_body, use_ln=use_ln),
        grid=(_GRID,),
        in_specs=[
            pl.BlockSpec((NC, _RB, D), lambda i: (0, i, 0)),
            pl.BlockSpec((_RB, D), lambda i: (i, 0)),
            pl.BlockSpec((1, D), lambda i: (0, 0)),
            pl.BlockSpec((1, D), lambda i: (0, 0)),
            pl.BlockSpec((1, D), lambda i: (0, 0)),
            pl.BlockSpec((D, D), lambda i: (0, 0)),
        ],
        out_specs=pl.BlockSpec((_RB, D), lambda i: (i, 0)),
        out_shape=jax.ShapeDtypeStruct((N, D), _f32),
    )(part, dinv_b, bias, gamma, beta, w_next)


def _final_body(p_ref, dinv_ref, bias_ref, o_ref):
    o_ref[...] = dinv_ref[...] * (p_ref[0] + p_ref[1]) + bias_ref[...]


def _tc_final(part, dinv_b, bias):
    return pl.pallas_call(
        _final_body,
        grid=(_GRID,),
        in_specs=[
            pl.BlockSpec((NC, _RB, D), lambda i: (0, i, 0)),
            pl.BlockSpec((_RB, D), lambda i: (i, 0)),
            pl.BlockSpec((1, D), lambda i: (0, 0)),
        ],
        out_specs=pl.BlockSpec((_RB, D), lambda i: (i, 0)),
        out_shape=jax.ShapeDtypeStruct((N, D), _f32),
    )(part, dinv_b, bias)


def kernel(x, hyperedge_index, W1, b1, W2, b2, W3, b3, W4, b4, W5, b5,
           gamma, beta):
    nidx = hyperedge_index[0]
    eidx = hyperedge_index[1]
    zeros_big = jnp.zeros((NPAD, D), _f32)
    ones_rows = jnp.ones((C, D), _f32)

    padn = EPTP - EPT
    t_ = jnp.arange(NT, dtype=jnp.int32)[:, None]
    j_ = jnp.arange(padn, dtype=jnp.int32)[None, :]
    gfill = (t_ * 613 + j_ * 97) % N
    sfill = N + (t_ * 7 + j_) % (NPAD - N)

    def _pad_idx(idx, fill):
        a = idx.reshape(NT, EPT)
        return jnp.concatenate([a, fill], axis=1).reshape(-1)

    nidx_g = _pad_idx(nidx, gfill)
    eidx_s = _pad_idx(eidx, sfill)
    eidx_g = _pad_idx(eidx, gfill)
    nidx_s = _pad_idx(nidx, sfill)

    dcnt, bcnt = _sc_counts(nidx, eidx, ones_rows, zeros_big)
    dinv_b, binv_b = _tc_inv(dcnt, bcnt)

    Ws = [W1, W2, W3, W4, W5]
    bs = [b.reshape(1, D) for b in (b1, b2, b3, b4, b5)]
    gamma2 = gamma.reshape(1, D)
    beta2 = beta.reshape(1, D)

    xw = _tc_mm(x, W1)
    for i in range(5):
        pA = _sc_stage_pipe(xw, nidx_g, eidx_s, zeros_big)
        ef = _tc_combine(pA, binv_b)
        pB = _sc_stage_pipe(ef, eidx_g, nidx_s, zeros_big)
        if i < 4:
            xw = _tc_post(pB, dinv_b, bs[i], gamma2, beta2, Ws[i + 1],
                          use_ln=(i == 0))
        else:
            z = _tc_final(pB, dinv_b, bs[4])
    return z


# speedup vs baseline: 4.2689x; 1.1288x over previous
import functools

import jax
import jax.numpy as jnp
from jax import lax
from jax.experimental import pallas as pl
from jax.experimental.pallas import tpu as pltpu
from jax.experimental.pallas import tpu_sc as plsc

N = 10000
E = 320000
D = 128
NPAD = 10240
NC = 2
NS = 16
NT = NC * NS
EPT = E // NT
C = 200
CHUNKS = EPT // C
STRIPE = NPAD // NS

_mesh = plsc.VectorSubcoreMesh(core_axis_name="c", subcore_axis_name="s")

_f32 = jnp.float32


@functools.partial(
    pl.kernel,
    mesh=_mesh,
    out_type=[
        jax.ShapeDtypeStruct((NC, NPAD, D), _f32),
        jax.ShapeDtypeStruct((NC, NPAD, D), _f32),
    ],
    scratch_types=[
        pltpu.VMEM((C,), jnp.int32),
        pltpu.VMEM((C, D), _f32),
        pltpu.VMEM_SHARED((NPAD, D), _f32),
    ],
)
def _sc_counts(nidx_hbm, eidx_hbm, ones_hbm, zeros_hbm,
               dcnt_hbm, bcnt_hbm, idx_v, ones_v, acc_sh):
    c = lax.axis_index("c")
    s = lax.axis_index("s")
    pltpu.sync_copy(ones_hbm, ones_v)
    base = (c * NS + s) * EPT

    for idx_hbm, out_hbm in ((nidx_hbm, dcnt_hbm), (eidx_hbm, bcnt_hbm)):
        pltpu.sync_copy(zeros_hbm.at[pl.ds(s * STRIPE, STRIPE)],
                        acc_sh.at[pl.ds(s * STRIPE, STRIPE)])
        plsc.subcore_barrier()

        @pl.loop(0, CHUNKS)
        def _(j):
            off = base + j * C
            pltpu.sync_copy(idx_hbm.at[pl.ds(off, C)], idx_v)
            pltpu.sync_copy(ones_v, acc_sh.at[idx_v], add=True)

        plsc.subcore_barrier()
        pltpu.sync_copy(acc_sh.at[pl.ds(s * STRIPE, STRIPE)],
                        out_hbm.at[c, pl.ds(s * STRIPE, STRIPE)])
        plsc.subcore_barrier()


CP = 256
CHP = 40
EPTP = CP * CHP
EPAD = NT * EPTP
DUMP = NPAD - 1


NCHK = NT * CHP


@functools.partial(
    pl.kernel,
    mesh=_mesh,
    out_type=jax.ShapeDtypeStruct((NC, NPAD, D), _f32),
    scratch_types=[
        pltpu.VMEM((CP, D), _f32),
        pltpu.VMEM_SHARED((NPAD, D), _f32),
        pltpu.SemaphoreType.DMA,
    ],
)
def _sc_stage_pipe(feat_hbm, gidx_hbm, sidx_hbm, zeros_hbm, out_hbm,
                   rows_v, acc_sh, sem):
    s = lax.axis_index("s")
    pltpu.sync_copy(zeros_hbm.at[pl.ds(s * STRIPE, STRIPE)],
                    acc_sh.at[pl.ds(s * STRIPE, STRIPE)])
    plsc.subcore_barrier()

    def body(gi_ref, si_ref):
        pltpu.async_copy(feat_hbm.at[gi_ref.at[0]], rows_v, sem).wait()
        pltpu.sync_copy(rows_v, acc_sh.at[si_ref.at[0]], add=True)

    pltpu.emit_pipeline(
        body,
        grid=(NCHK,),
        in_specs=[
            pl.BlockSpec((1, CP), lambda i: (0, i)),
            pl.BlockSpec((1, CP), lambda i: (0, i)),
        ],
        out_specs=[],
        core_axis_name=("c", "s"),
        dimension_semantics=(pltpu.PARALLEL,),
    )(gidx_hbm, sidx_hbm)

    plsc.subcore_barrier()
    c = lax.axis_index("c")
    pltpu.sync_copy(acc_sh.at[pl.ds(s * STRIPE, STRIPE)],
                    out_hbm.at[c, pl.ds(s * STRIPE, STRIPE)])


_RB = 400
_GRID = N // _RB


def _inv_body(d_ref, b_ref, dinv_ref, binv_ref):
    ds_ = d_ref[0, :, 0:1] + d_ref[1, :, 0:1]
    bs_ = b_ref[0, :, 0:1] + b_ref[1, :, 0:1]
    dinv = jnp.where(ds_ > 0, 1.0 / ds_, 0.0)
    binv = jnp.where(bs_ > 0, 1.0 / bs_, 0.0)
    dinv_ref[...] = jnp.broadcast_to(dinv, (512, D))
    binv_ref[...] = jnp.broadcast_to(binv, (512, D))


def _tc_inv(dcnt, bcnt):
    return pl.pallas_call(
        _inv_body,
        grid=(NPAD // 512,),
        in_specs=[
            pl.BlockSpec((NC, 512, D), lambda i: (0, i, 0)),
            pl.BlockSpec((NC, 512, D), lambda i: (0, i, 0)),
        ],
        out_specs=[
            pl.BlockSpec((512, D), lambda i: (i, 0)),
            pl.BlockSpec((512, D), lambda i: (i, 0)),
        ],
        out_shape=[
            jax.ShapeDtypeStruct((NPAD, D), _f32),
            jax.ShapeDtypeStruct((NPAD, D), _f32),
        ],
    )(dcnt, bcnt)


def _mm_body(x_ref, w_ref, o_ref):
    o_ref[...] = jnp.dot(x_ref[...], w_ref[...],
                         preferred_element_type=_f32)


def _tc_mm(x, w):
    return pl.pallas_call(
        _mm_body,
        grid=(_GRID,),
        in_specs=[
            pl.BlockSpec((_RB, D), lambda i: (i, 0)),
            pl.BlockSpec((D, D), lambda i: (0, 0)),
        ],
        out_specs=pl.BlockSpec((_RB, D), lambda i: (i, 0)),
        out_shape=jax.ShapeDtypeStruct((N, D), _f32),
    )(x, w)


def _combine_body(p_ref, binv_ref, o_ref):
    o_ref[...] = binv_ref[...] * (p_ref[0] + p_ref[1])


def _tc_combine(part, binv_b):
    return pl.pallas_call(
        _combine_body,
        grid=(_GRID,),
        in_specs=[
            pl.BlockSpec((NC, _RB, D), lambda i: (0, i, 0)),
            pl.BlockSpec((_RB, D), lambda i: (i, 0)),
        ],
        out_specs=pl.BlockSpec((_RB, D), lambda i: (i, 0)),
        out_shape=jax.ShapeDtypeStruct((N, D), _f32),
    )(part, binv_b)


def _post_body(p_ref, dinv_ref, bias_ref, gamma_ref, beta_ref, w_ref, o_ref,
               *, use_ln):
    h = dinv_ref[...] * (p_ref[0] + p_ref[1]) + bias_ref[...]
    if use_ln:
        mu = jnp.mean(h, axis=-1, keepdims=True)
        var = jnp.mean((h - mu) ** 2, axis=-1, keepdims=True)
        h = (h - mu) / jnp.sqrt(var + 1e-5) * gamma_ref[...] + beta_ref[...]
    h = jnp.maximum(h, 0.0)
    o_ref[...] = jnp.dot(h, w_ref[...], preferred_element_type=_f32)


def _tc_post(part, dinv_b, bias, gamma, beta, w_next, use_ln):
    return pl.pallas_call(
        functools.partial(_post_body, use_ln=use_ln),
        grid=(_GRID,),
        in_specs=[
            pl.BlockSpec((NC, _RB, D), lambda i: (0, i, 0)),
            pl.BlockSpec((_RB, D), lambda i: (i, 0)),
            pl.BlockSpec((1, D), lambda i: (0, 0)),
            pl.BlockSpec((1, D), lambda i: (0, 0)),
            pl.BlockSpec((1, D), lambda i: (0, 0)),
            pl.BlockSpec((D, D), lambda i: (0, 0)),
        ],
        out_specs=pl.BlockSpec((_RB, D), lambda i: (i, 0)),
        out_shape=jax.ShapeDtypeStruct((N, D), _f32),
    )(part, dinv_b, bias, gamma, beta, w_next)


def _final_body(p_ref, dinv_ref, bias_ref, o_ref):
    o_ref[...] = dinv_ref[...] * (p_ref[0] + p_ref[1]) + bias_ref[...]


def _tc_final(part, dinv_b, bias):
    return pl.pallas_call(
        _final_body,
        grid=(_GRID,),
        in_specs=[
            pl.BlockSpec((NC, _RB, D), lambda i: (0, i, 0)),
            pl.BlockSpec((_RB, D), lambda i: (i, 0)),
            pl.BlockSpec((1, D), lambda i: (0, 0)),
        ],
        out_specs=pl.BlockSpec((_RB, D), lambda i: (i, 0)),
        out_shape=jax.ShapeDtypeStruct((N, D), _f32),
    )(part, dinv_b, bias)


def kernel(x, hyperedge_index, W1, b1, W2, b2, W3, b3, W4, b4, W5, b5,
           gamma, beta):
    nidx = hyperedge_index[0]
    eidx = hyperedge_index[1]
    zeros_big = jnp.zeros((NPAD, D), _f32)
    ones_rows = jnp.ones((C, D), _f32)

    padn = EPTP - EPT
    t_ = jnp.arange(NT, dtype=jnp.int32)[:, None]
    j_ = jnp.arange(padn, dtype=jnp.int32)[None, :]
    gfill = (t_ * 613 + j_ * 97) % N
    sfill = N + (t_ * 7 + j_) % (NPAD - N)

    def _pad_idx(idx, fill):
        a = idx.reshape(NT, EPT)
        return jnp.concatenate([a, fill], axis=1).reshape(1, -1)

    nidx_g = _pad_idx(nidx, gfill)
    eidx_s = _pad_idx(eidx, sfill)
    eidx_g = _pad_idx(eidx, gfill)
    nidx_s = _pad_idx(nidx, sfill)

    dcnt, bcnt = _sc_counts(nidx, eidx, ones_rows, zeros_big)
    dinv_b, binv_b = _tc_inv(dcnt, bcnt)

    Ws = [W1, W2, W3, W4, W5]
    bs = [b.reshape(1, D) for b in (b1, b2, b3, b4, b5)]
    gamma2 = gamma.reshape(1, D)
    beta2 = beta.reshape(1, D)

    xw = _tc_mm(x, W1)
    for i in range(5):
        pA = _sc_stage_pipe(xw, nidx_g, eidx_s, zeros_big)
        ef = _tc_combine(pA, binv_b)
        pB = _sc_stage_pipe(ef, eidx_g, nidx_s, zeros_big)
        if i < 4:
            xw = _tc_post(pB, dinv_b, bs[i], gamma2, beta2, Ws[i + 1],
                          use_ln=(i == 0))
        else:
            z = _tc_final(pB, dinv_b, bs[4])
    return z


# speedup vs baseline: 5.2487x; 1.2295x over previous
import functools

import jax
import jax.numpy as jnp
from jax import lax
from jax.experimental import pallas as pl
from jax.experimental.pallas import tpu as pltpu
from jax.experimental.pallas import tpu_sc as plsc

N = 10000
E = 320000
D = 128
NPAD = 10240
NC = 2
NS = 16
NT = NC * NS
EPT = E // NT
C = 200
CHUNKS = EPT // C
STRIPE = NPAD // NS

_mesh = plsc.VectorSubcoreMesh(core_axis_name="c", subcore_axis_name="s")

_f32 = jnp.float32


@functools.partial(
    pl.kernel,
    mesh=_mesh,
    out_type=[
        jax.ShapeDtypeStruct((NC, NPAD, D), _f32),
        jax.ShapeDtypeStruct((NC, NPAD, D), _f32),
    ],
    scratch_types=[
        pltpu.VMEM((C,), jnp.int32),
        pltpu.VMEM((C, D), _f32),
        pltpu.VMEM_SHARED((NPAD, D), _f32),
    ],
)
def _sc_counts(nidx_hbm, eidx_hbm, ones_hbm, zeros_hbm,
               dcnt_hbm, bcnt_hbm, idx_v, ones_v, acc_sh):
    c = lax.axis_index("c")
    s = lax.axis_index("s")
    pltpu.sync_copy(ones_hbm, ones_v)
    base = (c * NS + s) * EPT

    for idx_hbm, out_hbm in ((nidx_hbm, dcnt_hbm), (eidx_hbm, bcnt_hbm)):
        pltpu.sync_copy(zeros_hbm.at[pl.ds(s * STRIPE, STRIPE)],
                        acc_sh.at[pl.ds(s * STRIPE, STRIPE)])
        plsc.subcore_barrier()

        @pl.loop(0, CHUNKS)
        def _(j):
            off = base + j * C
            pltpu.sync_copy(idx_hbm.at[pl.ds(off, C)], idx_v)
            pltpu.sync_copy(ones_v, acc_sh.at[idx_v], add=True)

        plsc.subcore_barrier()
        pltpu.sync_copy(acc_sh.at[pl.ds(s * STRIPE, STRIPE)],
                        out_hbm.at[c, pl.ds(s * STRIPE, STRIPE)])
        plsc.subcore_barrier()


SUB = 128
NSUB = 8
BLK = SUB * NSUB
BPT = 10
EPTP = BLK * BPT
CHP = EPTP // SUB
EPAD = NT * EPTP
NBLK = NT * BPT
DUMP = NPAD - 1


@functools.partial(
    pl.kernel,
    mesh=_mesh,
    out_type=jax.ShapeDtypeStruct((NC, NPAD, D), _f32),
    scratch_types=[
        pltpu.VMEM((SUB, D), _f32),
        pltpu.VMEM((SUB, D), _f32),
        pltpu.VMEM_SHARED((NPAD, D), _f32),
        pltpu.SemaphoreType.DMA,
        pltpu.SemaphoreType.DMA,
    ],
)
def _sc_stage_pipe(feat_hbm, gidx_hbm, sidx_hbm, zeros_hbm, out_hbm,
                   ra, rb, acc_sh, semg, sems):
    s = lax.axis_index("s")
    pltpu.sync_copy(zeros_hbm.at[pl.ds(s * STRIPE, STRIPE)],
                    acc_sh.at[pl.ds(s * STRIPE, STRIPE)])
    plsc.subcore_barrier()
    rbufs = (ra, rb)

    def body(gi_ref, si_ref):
        hs = [None] * NSUB
        hs[0] = pltpu.async_copy(feat_hbm.at[gi_ref.at[0, 0]], ra, semg)
        hs[1] = pltpu.async_copy(feat_hbm.at[gi_ref.at[0, 1]], rb, semg)
        for k in range(NSUB):
            r = rbufs[k % 2]
            hs[k].wait()
            sh = pltpu.async_copy(r, acc_sh.at[si_ref.at[0, k]], sems,
                                  add=True)
            sh.wait()
            if k + 2 < NSUB:
                hs[k + 2] = pltpu.async_copy(
                    feat_hbm.at[gi_ref.at[0, k + 2]], r, semg)

    pltpu.emit_pipeline(
        body,
        grid=(NBLK,),
        in_specs=[
            pl.BlockSpec((1, NSUB, SUB), lambda i: (i, 0, 0)),
            pl.BlockSpec((1, NSUB, SUB), lambda i: (i, 0, 0)),
        ],
        out_specs=[],
        core_axis_name=("c", "s"),
        dimension_semantics=(pltpu.PARALLEL,),
    )(gidx_hbm, sidx_hbm)

    plsc.subcore_barrier()
    c = lax.axis_index("c")
    pltpu.sync_copy(acc_sh.at[pl.ds(s * STRIPE, STRIPE)],
                    out_hbm.at[c, pl.ds(s * STRIPE, STRIPE)])


_RB = 400
_GRID = N // _RB


def _inv_body(d_ref, b_ref, dinv_ref, binv_ref):
    ds_ = d_ref[0, :, 0:1] + d_ref[1, :, 0:1]
    bs_ = b_ref[0, :, 0:1] + b_ref[1, :, 0:1]
    dinv = jnp.where(ds_ > 0, 1.0 / ds_, 0.0)
    binv = jnp.where(bs_ > 0, 1.0 / bs_, 0.0)
    dinv_ref[...] = jnp.broadcast_to(dinv, (512, D))
    binv_ref[...] = jnp.broadcast_to(binv, (512, D))


def _tc_inv(dcnt, bcnt):
    return pl.pallas_call(
        _inv_body,
        grid=(NPAD // 512,),
        in_specs=[
            pl.BlockSpec((NC, 512, D), lambda i: (0, i, 0)),
            pl.BlockSpec((NC, 512, D), lambda i: (0, i, 0)),
        ],
        out_specs=[
            pl.BlockSpec((512, D), lambda i: (i, 0)),
            pl.BlockSpec((512, D), lambda i: (i, 0)),
        ],
        out_shape=[
            jax.ShapeDtypeStruct((NPAD, D), _f32),
            jax.ShapeDtypeStruct((NPAD, D), _f32),
        ],
    )(dcnt, bcnt)


def _mm_body(x_ref, w_ref, o_ref):
    o_ref[...] = jnp.dot(x_ref[...], w_ref[...],
                         preferred_element_type=_f32)


def _tc_mm(x, w):
    return pl.pallas_call(
        _mm_body,
        grid=(_GRID,),
        in_specs=[
            pl.BlockSpec((_RB, D), lambda i: (i, 0)),
            pl.BlockSpec((D, D), lambda i: (0, 0)),
        ],
        out_specs=pl.BlockSpec((_RB, D), lambda i: (i, 0)),
        out_shape=jax.ShapeDtypeStruct((N, D), _f32),
    )(x, w)


def _combine_body(p_ref, binv_ref, o_ref):
    o_ref[...] = binv_ref[...] * (p_ref[0] + p_ref[1])


def _tc_combine(part, binv_b):
    return pl.pallas_call(
        _combine_body,
        grid=(_GRID,),
        in_specs=[
            pl.BlockSpec((NC, _RB, D), lambda i: (0, i, 0)),
            pl.BlockSpec((_RB, D), lambda i: (i, 0)),
        ],
        out_specs=pl.BlockSpec((_RB, D), lambda i: (i, 0)),
        out_shape=jax.ShapeDtypeStruct((N, D), _f32),
    )(part, binv_b)


def _post_body(p_ref, dinv_ref, bias_ref, gamma_ref, beta_ref, w_ref, o_ref,
               *, use_ln):
    h = dinv_ref[...] * (p_ref[0] + p_ref[1]) + bias_ref[...]
    if use_ln:
        mu = jnp.mean(h, axis=-1, keepdims=True)
        var = jnp.mean((h - mu) ** 2, axis=-1, keepdims=True)
        h = (h - mu) / jnp.sqrt(var + 1e-5) * gamma_ref[...] + beta_ref[...]
    h = jnp.maximum(h, 0.0)
    o_ref[...] = jnp.dot(h, w_ref[...], preferred_element_type=_f32)


def _tc_post(part, dinv_b, bias, gamma, beta, w_next, use_ln):
    return pl.pallas_call(
        functools.partial(_post_body, use_ln=use_ln),
        grid=(_GRID,),
        in_specs=[
            pl.BlockSpec((NC, _RB, D), lambda i: (0, i, 0)),
            pl.BlockSpec((_RB, D), lambda i: (i, 0)),
            pl.BlockSpec((1, D), lambda i: (0, 0)),
            pl.BlockSpec((1, D), lambda i: (0, 0)),
            pl.BlockSpec((1, D), lambda i: (0, 0)),
            pl.BlockSpec((D, D), lambda i: (0, 0)),
        ],
        out_specs=pl.BlockSpec((_RB, D), lambda i: (i, 0)),
        out_shape=jax.ShapeDtypeStruct((N, D), _f32),
    )(part, dinv_b, bias, gamma, beta, w_next)


def _final_body(p_ref, dinv_ref, bias_ref, o_ref):
    o_ref[...] = dinv_ref[...] * (p_ref[0] + p_ref[1]) + bias_ref[...]


def _tc_final(part, dinv_b, bias):
    return pl.pallas_call(
        _final_body,
        grid=(_GRID,),
        in_specs=[
            pl.BlockSpec((NC, _RB, D), lambda i: (0, i, 0)),
            pl.BlockSpec((_RB, D), lambda i: (i, 0)),
            pl.BlockSpec((1, D), lambda i: (0, 0)),
        ],
        out_specs=pl.BlockSpec((_RB, D), lambda i: (i, 0)),
        out_shape=jax.ShapeDtypeStruct((N, D), _f32),
    )(part, dinv_b, bias)


def kernel(x, hyperedge_index, W1, b1, W2, b2, W3, b3, W4, b4, W5, b5,
           gamma, beta):
    nidx = hyperedge_index[0]
    eidx = hyperedge_index[1]
    zeros_big = jnp.zeros((NPAD, D), _f32)
    ones_rows = jnp.ones((C, D), _f32)

    padn = EPTP - EPT
    t_ = jnp.arange(NT, dtype=jnp.int32)[:, None]
    j_ = jnp.arange(padn, dtype=jnp.int32)[None, :]
    gfill = (t_ * 613 + j_ * 97) % N
    sfill = N + (t_ * 7 + j_) % (NPAD - N)

    def _pad_idx(idx, fill):
        a = idx.reshape(NT, EPT)
        return jnp.concatenate([a, fill], axis=1).reshape(NBLK, NSUB, SUB)

    nidx_g = _pad_idx(nidx, gfill)
    eidx_s = _pad_idx(eidx, sfill)
    eidx_g = _pad_idx(eidx, gfill)
    nidx_s = _pad_idx(nidx, sfill)

    dcnt, bcnt = _sc_counts(nidx, eidx, ones_rows, zeros_big)
    dinv_b, binv_b = _tc_inv(dcnt, bcnt)

    Ws = [W1, W2, W3, W4, W5]
    bs = [b.reshape(1, D) for b in (b1, b2, b3, b4, b5)]
    gamma2 = gamma.reshape(1, D)
    beta2 = beta.reshape(1, D)

    xw = _tc_mm(x, W1)
    for i in range(5):
        pA = _sc_stage_pipe(xw, nidx_g, eidx_s, zeros_big)
        ef = _tc_combine(pA, binv_b)
        pB = _sc_stage_pipe(ef, eidx_g, nidx_s, zeros_big)
        if i < 4:
            xw = _tc_post(pB, dinv_b, bs[i], gamma2, beta2, Ws[i + 1],
                          use_ln=(i == 0))
        else:
            z = _tc_final(pB, dinv_b, bs[4])
    return z
